# baseline (speedup 1.0000x reference)
import functools

import jax
import jax.numpy as jnp
import numpy as np
from jax import lax
from jax.experimental import pallas as pl
from jax.experimental.pallas import tpu as pltpu
from jax.experimental.pallas import tpu_sc as plsc

_LAG = 24
_QN = 64
_B = 32
_T = 8192
_TOUT = _T - _LAG + 1
_PADT = 8256
_CHUNK = 512
_SC_ROWS = 8168
_NCH = -(-_SC_ROWS // _CHUNK)
_NI = _T // 16 + 1

_UTAB = np.full((128,), np.inf, dtype=np.float32)
_UTAB[0] = -np.inf
_UTAB[1:65] = np.linspace(-2.0, 2.0, _QN + 1)[:-1].astype(np.float32)

_mesh = plsc.VectorSubcoreMesh(core_axis_name="c", subcore_axis_name="s")


@functools.partial(
    pl.kernel,
    out_type=jax.ShapeDtypeStruct((_B, _TOUT * _QN), jnp.float32),
    mesh=_mesh,
    scratch_types=[
        pltpu.VMEM((_T * 4,), jnp.float32),
        pltpu.VMEM((_PADT,), jnp.float32),
        pltpu.VMEM((_PADT,), jnp.float32),
        pltpu.VMEM((_PADT,), jnp.float32),
        pltpu.VMEM((128,), jnp.float32),
        pltpu.VMEM((_CHUNK * _QN,), jnp.float32),
        pltpu.VMEM((_CHUNK * _QN,), jnp.float32),
        pltpu.VMEM((_CHUNK,), jnp.int32),
        pltpu.VMEM((_CHUNK,), jnp.int32),
        pltpu.SemaphoreType.DMA,
        pltpu.SemaphoreType.DMA,
    ],
    compiler_params=pltpu.CompilerParams(needs_layout_passes=False),
)
def _sc_onehot(x_hbm, u_hbm, out_hbm, xr, xv, wa, wb, ut, buf0, buf1, cb0,
               cb1, sem0, sem1):
    wid = lax.axis_index("s") * 2 + lax.axis_index("c")
    iota = lax.iota(jnp.int32, 16)
    ninf = jnp.full((16,), -jnp.inf, dtype=jnp.float32)
    ones = jnp.full((16,), 1.0, dtype=jnp.float32)
    zeros = jnp.full((16,), 0.0, dtype=jnp.float32)

    pltpu.sync_copy(u_hbm, ut)
    pltpu.sync_copy(x_hbm.at[wid], xr)

    def gbody(i, carry):
        b = i * 16
        idx = (b + iota) * 4 + 1
        xv[pl.ds(b, 16)] = plsc.load_gather(xr, [idx])
        return carry
    lax.fori_loop(0, _T // 16, gbody, 0)

    for b in range(_T, _PADT, 16):
        xv[pl.ds(b, 16)] = ninf
        wa[pl.ds(b, 16)] = ninf
        wb[pl.ds(b, 16)] = ninf

    def zbody(j, carry):
        buf0[pl.ds(j * 16, 16)] = zeros
        buf1[pl.ds(j * 16, 16)] = zeros
        return carry
    lax.fori_loop(0, _CHUNK * _QN // 16, zbody, 0)

    def mpass(dst, src, off):
        def body(i, carry):
            b = i * 16
            dst[pl.ds(b, 16)] = jnp.maximum(src[pl.ds(b, 16)],
                                            src[pl.ds(b + off, 16)])
            return carry
        lax.fori_loop(0, _NI, body, 0)

    mpass(wa, xv, 1)
    mpass(wb, wa, 2)
    mpass(wa, wb, 4)
    mpass(wb, wa, 8)

    def fbody(i, carry):
        b = i * 16
        xv[pl.ds(b, 16)] = jnp.maximum(wb[pl.ds(b, 16)], wa[pl.ds(b + 16, 16)])
        return carry
    lax.fori_loop(0, _NI, fbody, 0)

    bufs = (buf0, buf1)
    cbs = (cb0, cb1)
    sems = (sem0, sem1)
    copies = [None, None]

    for k in range(_NCH):
        base = k * _CHUNK
        rows = min(_CHUNK, _SC_ROWS - base)
        nj = -(-rows // 16)
        buf, cb, sem = bufs[k % 2], cbs[k % 2], sems[k % 2]

        if copies[k % 2] is not None:
            copies[k % 2].wait()
            def cbody(j, carry):
                b0 = j * 16
                plsc.store_scatter(buf, [cb[pl.ds(b0, 16)]], zeros)
                return carry
            lax.fori_loop(0, _CHUNK // 16, cbody, 0)

        def sbody(j, carry):
            b0 = j * 16
            t = base + b0 + iota
            v = xv[pl.ds(base + b0, 16)]
            u = jnp.clip((v + 2.0) * 16.0, -1.0, 64.0)
            c = (u + 1.0).astype(jnp.int32) - 1
            c = jnp.minimum(c, 63)
            ulo = plsc.load_gather(ut, [c + 1])
            uhi = plsc.load_gather(ut, [c + 2])
            c = jnp.where(v < ulo, c - 1, jnp.where(v >= uhi, c + 1, c))
            c_safe = jnp.maximum(c, 0)
            mask = (c >= 0) & (t < _TOUT)
            flat = (b0 + iota) * _QN + c_safe
            plsc.store_scatter(buf, [flat], ones, mask=mask)
            cb[pl.ds(b0, 16)] = flat
            return carry
        lax.fori_loop(0, nj, sbody, 0)

        copies[k % 2] = pltpu.async_copy(
            buf.at[pl.ds(0, rows * _QN)],
            out_hbm.at[wid, pl.ds(base * _QN, rows * _QN)], sem)

    copies[_NCH % 2].wait()
    copies[(_NCH + 1) % 2].wait()


def _tc_fixup_body(x_ref, aliased_ref, out_ref):
    del aliased_ref
    m = jnp.max(x_ref[:, 104:128, 1], axis=1, keepdims=True)
    ki = lax.broadcasted_iota(jnp.int32, (1, 128), 1)
    k = ki.astype(jnp.float32)
    lo = k * 0.0625 - 2.0
    hi = jnp.where(ki == _QN - 1, jnp.inf, lo + 0.0625)
    oh = (m >= lo).astype(jnp.float32) - (m >= hi).astype(jnp.float32)
    out_ref[...] = jnp.where(ki < _QN, oh, 0.0)


def _tc_fixup(x, sc_out):
    return pl.pallas_call(
        _tc_fixup_body,
        out_shape=jax.ShapeDtypeStruct((_B, _TOUT * _QN), jnp.float32),
        grid=(1,),
        in_specs=[
            pl.BlockSpec((_B, 128, 4), lambda i: (0, _T // 128 - 1, 0)),
            pl.BlockSpec(memory_space=pltpu.MemorySpace.HBM),
        ],
        out_specs=pl.BlockSpec((_B, 128), lambda i: (0, _SC_ROWS * _QN // 128)),
        input_output_aliases={1: 0},
    )(x, sc_out)


def kernel(inp):
    out = _sc_onehot(inp.reshape(_B, _T * 4), jnp.asarray(_UTAB))
    out = _tc_fixup(inp, out)
    return out.reshape(_B, _TOUT, _QN)


# speedup vs baseline: 1.5390x; 1.5390x over previous
import functools

import jax
import jax.numpy as jnp
import numpy as np
from jax import lax
from jax.experimental import pallas as pl
from jax.experimental.pallas import tpu as pltpu
from jax.experimental.pallas import tpu_sc as plsc

_LAG = 24
_QN = 64
_B = 32
_T = 8192
_TOUT = _T - _LAG + 1
_PADT = 8256
_NI = _T // 16 + 1
_TB = 8176

_UTAB = np.full((128,), np.inf, dtype=np.float32)
_UTAB[0] = -np.inf
_UTAB[1:65] = np.linspace(-2.0, 2.0, _QN + 1)[:-1].astype(np.float32)

_mesh = plsc.VectorSubcoreMesh(core_axis_name="c", subcore_axis_name="s")


@functools.partial(
    pl.kernel,
    out_type=jax.ShapeDtypeStruct((_B, 1, _T), jnp.int32),
    mesh=_mesh,
    scratch_types=[
        pltpu.VMEM((_T * 4,), jnp.float32),
        pltpu.VMEM((_PADT,), jnp.float32),
        pltpu.VMEM((_PADT,), jnp.float32),
        pltpu.VMEM((_PADT,), jnp.float32),
        pltpu.VMEM((_T,), jnp.int32),
    ],
    compiler_params=pltpu.CompilerParams(needs_layout_passes=False),
)
def _sc_bins(x_hbm, out_hbm, xr, xv, wa, wb, bv):
    wid = lax.axis_index("s") * 2 + lax.axis_index("c")
    iota = lax.iota(jnp.int32, 16)
    ninf = jnp.full((16,), -jnp.inf, dtype=jnp.float32)

    pltpu.sync_copy(x_hbm.at[wid], xr)

    def gbody(i, carry):
        b = i * 16
        xv[pl.ds(b, 16)] = plsc.load_gather(xr, [(b + iota) * 4 + 1])
        return carry
    lax.fori_loop(0, _T // 16, gbody, 0)

    for b in range(_T, _PADT, 16):
        xv[pl.ds(b, 16)] = ninf
        wa[pl.ds(b, 16)] = ninf
        wb[pl.ds(b, 16)] = ninf

    def mpass(dst, src, off):
        def body(i, carry):
            b = i * 16
            dst[pl.ds(b, 16)] = jnp.maximum(src[pl.ds(b, 16)],
                                            src[pl.ds(b + off, 16)])
            return carry
        lax.fori_loop(0, _NI, body, 0)

    mpass(wa, xv, 1)
    mpass(wb, wa, 2)
    mpass(wa, wb, 4)
    mpass(wb, wa, 8)

    def fbody(i, carry):
        b = i * 16
        v = jnp.maximum(wb[pl.ds(b, 16)], wa[pl.ds(b + 16, 16)])
        u = jnp.clip((v + 2.0) * 16.0, -1.0, 64.0)
        c0 = (u + 1.0).astype(jnp.int32) - 1
        blo = c0.astype(jnp.float32) * 0.0625 - 2.0
        bhi = (c0 + 1).astype(jnp.float32) * 0.0625 - 2.0
        c = c0 - (v < blo).astype(jnp.int32) + (v >= bhi).astype(jnp.int32)
        bv[pl.ds(b, 16)] = jnp.clip(c, -1, 63)
        return carry
    lax.fori_loop(0, _T // 16, fbody, 0)

    pltpu.sync_copy(bv, out_hbm.at[wid, 0])


_CH = 1024
_NCHT = 8


def _tc_expand_body(bins_ref, out_ref, buf, sems):
    b = pl.program_id(0)
    row = bins_ref[0, 0, :]
    col = lax.broadcasted_iota(jnp.int32, (1, _QN), 1)
    for i in range(_NCHT):
        r0 = i * _CH
        nr = min(_CH, _TOUT - r0)
        cp = pltpu.make_async_copy(
            buf.at[i, pl.ds(0, nr)], out_ref.at[b, pl.ds(r0, nr)], sems.at[i])

        @pl.when(b > 0)
        def _wait_prev():
            pltpu.make_async_copy(
                buf.at[i, pl.ds(0, nr)],
                out_ref.at[b - 1, pl.ds(r0, nr)], sems.at[i]).wait()

        c = lax.slice(row, (r0,), (r0 + _CH,)).reshape(_CH, 1)
        buf[i] = (c == col).astype(jnp.float32)
        cp.start()

    @pl.when(b == _B - 1)
    def _drain():
        for i in range(_NCHT):
            r0 = i * _CH
            nr = min(_CH, _TOUT - r0)
            pltpu.make_async_copy(
                buf.at[i, pl.ds(0, nr)], out_ref.at[b, pl.ds(r0, nr)],
                sems.at[i]).wait()


def _tc_expand(bins):
    return pl.pallas_call(
        _tc_expand_body,
        out_shape=jax.ShapeDtypeStruct((_B, _TOUT, _QN), jnp.float32),
        grid=(_B,),
        in_specs=[pl.BlockSpec((1, 1, _T), lambda b: (b, 0, 0))],
        out_specs=pl.BlockSpec(memory_space=pltpu.MemorySpace.HBM),
        scratch_shapes=[
            pltpu.VMEM((_NCHT, _CH, _QN), jnp.float32),
            pltpu.SemaphoreType.DMA((_NCHT,)),
        ],
    )(bins)


def kernel(inp):
    bins = _sc_bins(inp.reshape(_B, _T * 4))
    return _tc_expand(bins)
